# baseline (speedup 1.0000x reference)
import functools

import jax
import jax.numpy as jnp
from jax import lax
from jax.experimental import pallas as pl
from jax.experimental.pallas import tpu as pltpu
from jax.experimental.pallas import tpu_sc as plsc

N_USERS = 25000
N_ITEMS = 25000
LATENT_DIM = 64
N_LAYERS = 3
N_EDGES = 800000
BATCH = 4096

NC = 2
NS = 16

HALF_PAD = 25088
DUMMY = 25000
NTBL = 2 * HALF_PAD
ROWS_PER_TILE = HALF_PAD // NS

CHUNK = 128
STAGE = 10
NBUF = 3
EHALF = 430080
CHPC = EHALF // CHUNK
CHUNKS_PER_TILE = CHPC // NS
STAGES_PER_TILE = CHUNKS_PER_TILE // STAGE

BPT = BATCH // (NC * NS)

_mesh = plsc.VectorSubcoreMesh(
    core_axis_name="c", subcore_axis_name="s", num_cores=NC, num_subcores=NS)


def _layer_body(tbl_in, srch, dstlh, valsh, out_tbl,
                acc, src_v, dstl_v, vals_v, rows0, rows1, rows2, gsem, ssem):
    c = lax.axis_index("c")
    s = lax.axis_index("s")
    rows = (rows0, rows1, rows2)

    zeros16 = jnp.zeros((16,), jnp.float32)

    def zrow(i, carry):
        for u in range(4):
            rows0[i, pl.ds(u * 16, 16)] = zeros16
            rows1[i, pl.ds(u * 16, 16)] = zeros16
        return carry

    lax.fori_loop(0, CHUNK, zrow, 0)
    zbase = s * ROWS_PER_TILE
    zcps = []
    for k in range(ROWS_PER_TILE // (2 * CHUNK)):
        zcps.append(pltpu.async_copy(
            rows0, acc.at[pl.ds(zbase + 2 * k * CHUNK, CHUNK)], gsem))
        zcps.append(pltpu.async_copy(
            rows1, acc.at[pl.ds(zbase + (2 * k + 1) * CHUNK, CHUNK)], ssem))
    rem = ROWS_PER_TILE % (2 * CHUNK)
    if rem:
        zcps.append(pltpu.async_copy(
            rows0.at[pl.ds(0, rem)],
            acc.at[pl.ds(zbase + ROWS_PER_TILE - rem, rem)], gsem))
    for cp in zcps:
        cp.wait()
    plsc.subcore_barrier()

    def scale_buf(buf, voff):
        @plsc.parallel_loop(0, CHUNK // 16, 1, unroll=2)
        def _scale(g):
            vv = vals_v[pl.ds(voff + g * 16, 16)]
            for j in range(16):
                e = g * 16 + j
                v = vv[j]
                for u in range(4):
                    buf[e, pl.ds(u * 16, 16)] = buf[e, pl.ds(u * 16, 16)] * v

    def stage_body(st, carry):
        cb = c * CHPC + s * CHUNKS_PER_TILE + st * STAGE
        pltpu.sync_copy(srch.at[pl.ds(cb, STAGE)], src_v)
        pltpu.sync_copy(dstlh.at[pl.ds(cb, STAGE)], dstl_v)
        pltpu.sync_copy(valsh.at[pl.ds(cb * CHUNK, STAGE * CHUNK)], vals_v)
        gcp = [None] * NBUF
        scp = [None] * NBUF
        for j in range(NBUF - 1):
            gcp[j] = pltpu.async_copy(tbl_in.at[src_v.at[j]], rows[j], gsem)
        for j in range(STAGE):
            b = j % NBUF
            nb = (j + NBUF - 1) % NBUF
            if j + NBUF - 1 < STAGE:
                if scp[nb] is not None:
                    scp[nb].wait()
                gcp[nb] = pltpu.async_copy(
                    tbl_in.at[src_v.at[j + NBUF - 1]], rows[nb], gsem)
            gcp[b].wait()
            scale_buf(rows[b], j * CHUNK)
            scp[b] = pltpu.async_copy(rows[b], acc.at[dstl_v.at[j]], ssem,
                                      add=True)
        for k in range(NBUF):
            if scp[(STAGE - 1 - k) % NBUF] is not None:
                scp[(STAGE - 1 - k) % NBUF].wait()
                scp[(STAGE - 1 - k) % NBUF] = None
        return carry

    lax.fori_loop(0, STAGES_PER_TILE, stage_body, 0)
    plsc.subcore_barrier()

    pltpu.sync_copy(acc.at[pl.ds(s * ROWS_PER_TILE, ROWS_PER_TILE)],
                    out_tbl.at[pl.ds(c * HALF_PAD + s * ROWS_PER_TILE,
                                     ROWS_PER_TILE)])


_layer_kernel = functools.partial(
    pl.kernel,
    out_type=jax.ShapeDtypeStruct((NTBL, LATENT_DIM), jnp.float32),
    mesh=_mesh,
    compiler_params=pltpu.CompilerParams(use_tc_tiling_on_sc=False),
    scratch_types=[
        pltpu.VMEM_SHARED((HALF_PAD, LATENT_DIM), jnp.float32),
        pltpu.VMEM((STAGE, CHUNK), jnp.int32),
        pltpu.VMEM((STAGE, CHUNK), jnp.int32),
        pltpu.VMEM((STAGE * CHUNK,), jnp.float32),
        pltpu.VMEM((CHUNK, LATENT_DIM), jnp.float32),
        pltpu.VMEM((CHUNK, LATENT_DIM), jnp.float32),
        pltpu.VMEM((CHUNK, LATENT_DIM), jnp.float32),
        pltpu.SemaphoreType.DMA,
        pltpu.SemaphoreType.DMA,
    ],
)(_layer_body)


def _gamma_body(t0, t1, t2, t3, uidxh, iidxh, gout,
                uv, iv, ub0, ub1, ub2, ub3, ib0, ib1, ib2, ib3, gv, sem):
    c = lax.axis_index("c")
    s = lax.axis_index("s")
    w = c * NS + s

    pltpu.sync_copy(uidxh.at[w], uv)
    pltpu.sync_copy(iidxh.at[w], iv)
    cps = [pltpu.async_copy(t0.at[uv], ub0, sem),
           pltpu.async_copy(t1.at[uv], ub1, sem),
           pltpu.async_copy(t2.at[uv], ub2, sem),
           pltpu.async_copy(t3.at[uv], ub3, sem),
           pltpu.async_copy(t0.at[iv], ib0, sem),
           pltpu.async_copy(t1.at[iv], ib1, sem),
           pltpu.async_copy(t2.at[iv], ib2, sem),
           pltpu.async_copy(t3.at[iv], ib3, sem)]
    for cp in cps:
        cp.wait()

    lanes = lax.iota(jnp.int32, 16)

    def dot_group(grp, carry):
        out = jnp.zeros((16,), jnp.float32)
        for j in range(16):
            e = grp * 16 + j
            psum = jnp.zeros((16,), jnp.float32)
            for u in range(4):
                sl = pl.ds(u * 16, 16)
                ua = ub0[e, sl] + ub1[e, sl] + ub2[e, sl] + ub3[e, sl]
                ia = ib0[e, sl] + ib1[e, sl] + ib2[e, sl] + ib3[e, sl]
                psum = psum + ua * ia
            tot = psum[0]
            for k in range(1, 16):
                tot = tot + psum[k]
            g = tot * jnp.float32(1.0 / 16.0)
            out = jnp.where(lanes == j, jnp.full((16,), g, jnp.float32), out)
        gv[pl.ds(grp * 16, 16)] = out
        return carry

    lax.fori_loop(0, BPT // 16, dot_group, 0)
    pltpu.sync_copy(gv, gout.at[pl.ds(w * BPT, BPT)])


_gamma_kernel = functools.partial(
    pl.kernel,
    out_type=jax.ShapeDtypeStruct((BATCH,), jnp.float32),
    mesh=_mesh,
    compiler_params=pltpu.CompilerParams(use_tc_tiling_on_sc=False),
    scratch_types=[
        pltpu.VMEM((BPT,), jnp.int32),
        pltpu.VMEM((BPT,), jnp.int32),
    ] + [pltpu.VMEM((BPT, LATENT_DIM), jnp.float32)] * 8
    + [
        pltpu.VMEM((BPT,), jnp.float32),
        pltpu.SemaphoreType.DMA,
    ],
)(_gamma_body)


def kernel(users, items, edge_index, edge_vals, user_emb, item_emb):
    src = edge_index[0].astype(jnp.int32)
    dst = edge_index[1].astype(jnp.int32)

    src_adj = src + jnp.where(src >= N_USERS, HALF_PAD - N_USERS, 0).astype(jnp.int32)

    half = (dst >= N_USERS).astype(jnp.int32)
    perm = jnp.argsort((half << 16) | src_adj)
    src_s = src_adj[perm]
    dst_s = dst[perm]
    vals_s = edge_vals.astype(jnp.float32)[perm]
    dstl_s = jnp.where(dst_s < N_USERS, dst_s, dst_s - N_USERS)
    n0 = jnp.sum(1 - half)

    ar = jnp.arange(EHALF, dtype=jnp.int32)
    spread = DUMMY + (ar % (HALF_PAD - DUMMY))
    pos0 = jnp.minimum(ar, N_EDGES - 1)
    ok0 = ar < n0
    pos1 = jnp.minimum(n0 + ar, N_EDGES - 1)
    ok1 = (n0 + ar) < N_EDGES
    srch = jnp.concatenate([
        jnp.where(ok0, src_s[pos0], 0),
        jnp.where(ok1, src_s[pos1], 0),
    ]).reshape(2 * CHPC, CHUNK)
    dstlh = jnp.concatenate([
        jnp.where(ok0, dstl_s[pos0], spread),
        jnp.where(ok1, dstl_s[pos1], spread),
    ]).reshape(2 * CHPC, CHUNK)
    valsh = jnp.concatenate([
        jnp.where(ok0, vals_s[pos0], 0.0),
        jnp.where(ok1, vals_s[pos1], 0.0),
    ])

    tbl0 = jnp.zeros((NTBL, LATENT_DIM), jnp.float32)
    tbl0 = tbl0.at[0:N_USERS].set(user_emb.astype(jnp.float32))
    tbl0 = tbl0.at[HALF_PAD:HALF_PAD + N_ITEMS].set(item_emb.astype(jnp.float32))

    t1 = _layer_kernel(tbl0, srch, dstlh, valsh)
    t2 = _layer_kernel(t1, srch, dstlh, valsh)
    t3 = _layer_kernel(t2, srch, dstlh, valsh)

    uidx = users.astype(jnp.int32).reshape(NC * NS, BPT)
    iidx = (items.astype(jnp.int32) + HALF_PAD).reshape(NC * NS, BPT)
    gamma = _gamma_kernel(tbl0, t1, t2, t3, uidx, iidx)
    return gamma


# speedup vs baseline: 1.6825x; 1.6825x over previous
import functools

import jax
import jax.numpy as jnp
from jax import lax
from jax.experimental import pallas as pl
from jax.experimental.pallas import tpu as pltpu
from jax.experimental.pallas import tpu_sc as plsc

N_USERS = 25000
N_ITEMS = 25000
LATENT_DIM = 64
N_LAYERS = 3
N_EDGES = 800000
BATCH = 4096

NC = 2
NS = 16

HALF_PAD = 25088
DUMMY = 25000
NTBL = 2 * HALF_PAD
ROWS_PER_TILE = HALF_PAD // NS

CHUNK = 128
STAGE = 10
NBUF = 3
E_PAD = 819200
CHPC = E_PAD // CHUNK
CHUNKS_PER_TILE = CHPC // NS
STAGES_PER_TILE = CHUNKS_PER_TILE // STAGE

BPT = BATCH // (NC * NS)

_mesh = plsc.VectorSubcoreMesh(
    core_axis_name="c", subcore_axis_name="s", num_cores=NC, num_subcores=NS)


def _layer_body(tbl_in, srch, dstlh, valsh, out_tbl,
                acc, src_v, dstl_v, vals_v, rows0, rows1, rows2, gsem, ssem):
    c = lax.axis_index("c")
    s = lax.axis_index("s")
    rows = (rows0, rows1, rows2)

    zeros16 = jnp.zeros((16,), jnp.float32)

    def zrow(i, carry):
        for u in range(4):
            rows0[i, pl.ds(u * 16, 16)] = zeros16
            rows1[i, pl.ds(u * 16, 16)] = zeros16
        return carry

    lax.fori_loop(0, CHUNK, zrow, 0)
    zbase = s * ROWS_PER_TILE
    zcps = []
    for k in range(ROWS_PER_TILE // (2 * CHUNK)):
        zcps.append(pltpu.async_copy(
            rows0, acc.at[pl.ds(zbase + 2 * k * CHUNK, CHUNK)], gsem))
        zcps.append(pltpu.async_copy(
            rows1, acc.at[pl.ds(zbase + (2 * k + 1) * CHUNK, CHUNK)], ssem))
    rem = ROWS_PER_TILE % (2 * CHUNK)
    if rem:
        zcps.append(pltpu.async_copy(
            rows0.at[pl.ds(0, rem)],
            acc.at[pl.ds(zbase + ROWS_PER_TILE - rem, rem)], gsem))
    for cp in zcps:
        cp.wait()
    plsc.subcore_barrier()

    def scale_buf(buf, voff):
        @plsc.parallel_loop(0, CHUNK // 16, 1, unroll=2)
        def _scale(g):
            vv = vals_v[pl.ds(voff + g * 16, 16)]
            for j in range(16):
                e = g * 16 + j
                v = vv[j]
                for u in range(4):
                    buf[e, pl.ds(u * 16, 16)] = buf[e, pl.ds(u * 16, 16)] * v

    def stage_body(st, carry):
        cb = s * CHUNKS_PER_TILE + st * STAGE
        pltpu.sync_copy(srch.at[pl.ds(cb, STAGE)], src_v)
        pltpu.sync_copy(dstlh.at[pl.ds(c * CHPC + cb, STAGE)], dstl_v)
        pltpu.sync_copy(valsh.at[pl.ds(cb * CHUNK, STAGE * CHUNK)], vals_v)
        gcp = [None] * NBUF
        scp = [None] * NBUF
        for j in range(NBUF - 1):
            gcp[j] = pltpu.async_copy(tbl_in.at[src_v.at[j]], rows[j], gsem)
        for j in range(STAGE):
            b = j % NBUF
            nb = (j + NBUF - 1) % NBUF
            if j + NBUF - 1 < STAGE:
                if scp[nb] is not None:
                    scp[nb].wait()
                gcp[nb] = pltpu.async_copy(
                    tbl_in.at[src_v.at[j + NBUF - 1]], rows[nb], gsem)
            gcp[b].wait()
            scale_buf(rows[b], j * CHUNK)
            scp[b] = pltpu.async_copy(rows[b], acc.at[dstl_v.at[j]], ssem,
                                      add=True)
        for k in range(NBUF):
            if scp[(STAGE - 1 - k) % NBUF] is not None:
                scp[(STAGE - 1 - k) % NBUF].wait()
                scp[(STAGE - 1 - k) % NBUF] = None
        return carry

    lax.fori_loop(0, STAGES_PER_TILE, stage_body, 0)
    plsc.subcore_barrier()

    pltpu.sync_copy(acc.at[pl.ds(s * ROWS_PER_TILE, ROWS_PER_TILE)],
                    out_tbl.at[pl.ds(c * HALF_PAD + s * ROWS_PER_TILE,
                                     ROWS_PER_TILE)])


_layer_kernel = functools.partial(
    pl.kernel,
    out_type=jax.ShapeDtypeStruct((NTBL, LATENT_DIM), jnp.float32),
    mesh=_mesh,
    compiler_params=pltpu.CompilerParams(use_tc_tiling_on_sc=False),
    scratch_types=[
        pltpu.VMEM_SHARED((HALF_PAD, LATENT_DIM), jnp.float32),
        pltpu.VMEM((STAGE, CHUNK), jnp.int32),
        pltpu.VMEM((STAGE, CHUNK), jnp.int32),
        pltpu.VMEM((STAGE * CHUNK,), jnp.float32),
        pltpu.VMEM((CHUNK, LATENT_DIM), jnp.float32),
        pltpu.VMEM((CHUNK, LATENT_DIM), jnp.float32),
        pltpu.VMEM((CHUNK, LATENT_DIM), jnp.float32),
        pltpu.SemaphoreType.DMA,
        pltpu.SemaphoreType.DMA,
    ],
)(_layer_body)


def _gamma_body(t0, t1, t2, t3, uidxh, iidxh, gout,
                uv, iv, ub0, ub1, ub2, ub3, ib0, ib1, ib2, ib3, gv, sem):
    c = lax.axis_index("c")
    s = lax.axis_index("s")
    w = c * NS + s

    pltpu.sync_copy(uidxh.at[w], uv)
    pltpu.sync_copy(iidxh.at[w], iv)
    cps = [pltpu.async_copy(t0.at[uv], ub0, sem),
           pltpu.async_copy(t1.at[uv], ub1, sem),
           pltpu.async_copy(t2.at[uv], ub2, sem),
           pltpu.async_copy(t3.at[uv], ub3, sem),
           pltpu.async_copy(t0.at[iv], ib0, sem),
           pltpu.async_copy(t1.at[iv], ib1, sem),
           pltpu.async_copy(t2.at[iv], ib2, sem),
           pltpu.async_copy(t3.at[iv], ib3, sem)]
    for cp in cps:
        cp.wait()

    lanes = lax.iota(jnp.int32, 16)

    def dot_group(grp, carry):
        out = jnp.zeros((16,), jnp.float32)
        for j in range(16):
            e = grp * 16 + j
            psum = jnp.zeros((16,), jnp.float32)
            for u in range(4):
                sl = pl.ds(u * 16, 16)
                ua = ub0[e, sl] + ub1[e, sl] + ub2[e, sl] + ub3[e, sl]
                ia = ib0[e, sl] + ib1[e, sl] + ib2[e, sl] + ib3[e, sl]
                psum = psum + ua * ia
            tot = psum[0]
            for k in range(1, 16):
                tot = tot + psum[k]
            g = tot * jnp.float32(1.0 / 16.0)
            out = jnp.where(lanes == j, jnp.full((16,), g, jnp.float32), out)
        gv[pl.ds(grp * 16, 16)] = out
        return carry

    lax.fori_loop(0, BPT // 16, dot_group, 0)
    pltpu.sync_copy(gv, gout.at[pl.ds(w * BPT, BPT)])


_gamma_kernel = functools.partial(
    pl.kernel,
    out_type=jax.ShapeDtypeStruct((BATCH,), jnp.float32),
    mesh=_mesh,
    compiler_params=pltpu.CompilerParams(use_tc_tiling_on_sc=False),
    scratch_types=[
        pltpu.VMEM((BPT,), jnp.int32),
        pltpu.VMEM((BPT,), jnp.int32),
    ] + [pltpu.VMEM((BPT, LATENT_DIM), jnp.float32)] * 8
    + [
        pltpu.VMEM((BPT,), jnp.float32),
        pltpu.SemaphoreType.DMA,
    ],
)(_gamma_body)


def kernel(users, items, edge_index, edge_vals, user_emb, item_emb):
    src = edge_index[0].astype(jnp.int32)
    dst = edge_index[1].astype(jnp.int32)

    src_adj = src + jnp.where(src >= N_USERS, HALF_PAD - N_USERS, 0).astype(jnp.int32)

    pad_e = E_PAD - N_EDGES
    srch = jnp.concatenate(
        [src_adj, jnp.zeros((pad_e,), jnp.int32)]).reshape(CHPC, CHUNK)

    spread = DUMMY + (jnp.arange(N_EDGES, dtype=jnp.int32) % (HALF_PAD - DUMMY))
    dstl0 = jnp.where(dst < N_USERS, dst, spread)
    dstl1 = jnp.where(dst >= N_USERS, dst - N_USERS, spread)
    pad_d = DUMMY + (jnp.arange(pad_e, dtype=jnp.int32) % (HALF_PAD - DUMMY))
    dstlh = jnp.concatenate([
        jnp.concatenate([dstl0, pad_d]),
        jnp.concatenate([dstl1, pad_d]),
    ]).reshape(2 * CHPC, CHUNK)

    valsh = jnp.concatenate(
        [edge_vals.astype(jnp.float32), jnp.zeros((pad_e,), jnp.float32)])

    tbl0 = jnp.zeros((NTBL, LATENT_DIM), jnp.float32)
    tbl0 = tbl0.at[0:N_USERS].set(user_emb.astype(jnp.float32))
    tbl0 = tbl0.at[HALF_PAD:HALF_PAD + N_ITEMS].set(item_emb.astype(jnp.float32))

    t1 = _layer_kernel(tbl0, srch, dstlh, valsh)
    t2 = _layer_kernel(t1, srch, dstlh, valsh)
    t3 = _layer_kernel(t2, srch, dstlh, valsh)

    uidx = users.astype(jnp.int32).reshape(NC * NS, BPT)
    iidx = (items.astype(jnp.int32) + HALF_PAD).reshape(NC * NS, BPT)
    gamma = _gamma_kernel(tbl0, t1, t2, t3, uidx, iidx)
    return gamma
